# Initial kernel scaffold; baseline (speedup 1.0000x reference)
import jax
import jax.numpy as jnp
from jax.experimental import pallas as pl


def kernel(x, edge_index, edge_weight, W0, b0, W1, b1, W2, b2):
    raise NotImplementedError("write your pallas kernel here")


# speedup vs baseline: 3.9991x; 3.9991x over previous
import functools

import jax
import jax.numpy as jnp
from jax import lax
from jax.experimental import pallas as pl
from jax.experimental.pallas import tpu as pltpu
from jax.experimental.pallas import tpu_sc as plsc

_N = 10000
_E = 320000
_NSC = 2
_NTILE = 16
_NW = _NSC * _NTILE
_EW = _E // _NW
_C = 80
_RPT = 624
_TAIL = _N - _NTILE * _RPT
_ZC = 208


def _spmm_body(D, m_hbm, col_hbm, row_hbm, w_hbm, out_hbm,
               col_v, row_v, w_v, rows_v, zbuf, acc, sem):
    c = lax.axis_index("c")
    s = lax.axis_index("s")
    wid = c * _NTILE + s

    def zrow(i, carry):
        for j in range(D // 16):
            zbuf[i, pl.ds(j * 16, 16)] = jnp.zeros((16,), jnp.float32)
        return carry

    lax.fori_loop(0, _ZC, zrow, 0)
    for k in range(_RPT // _ZC):
        pltpu.sync_copy(zbuf, acc.at[pl.ds(s * _RPT + k * _ZC, _ZC)])

    @pl.when(s == _NTILE - 1)
    def _zero_tail():
        pltpu.sync_copy(zbuf.at[pl.ds(0, _TAIL)],
                        acc.at[pl.ds(_NTILE * _RPT, _TAIL)])

    plsc.subcore_barrier()

    base0 = wid * _EW

    def chunk_body(k, carry):
        base = base0 + k * _C
        pltpu.sync_copy(col_hbm.at[pl.ds(base, _C)], col_v)
        pltpu.sync_copy(row_hbm.at[pl.ds(base, _C)], row_v)
        pltpu.sync_copy(w_hbm.at[pl.ds(base, _C)], w_v)
        pltpu.async_copy(m_hbm.at[col_v], rows_v, sem).wait()

        def scale(g, inner):
            w16 = w_v[pl.ds(g * 16, 16)]
            for l in range(16):
                e = g * 16 + l
                wspl = jnp.full((16,), w16[l], jnp.float32)
                for j in range(D // 16):
                    rows_v[e, pl.ds(j * 16, 16)] = (
                        rows_v[e, pl.ds(j * 16, 16)] * wspl)
            return inner

        lax.fori_loop(0, _C // 16, scale, 0)
        pltpu.sync_copy(rows_v, acc.at[row_v], add=True)
        return carry

    lax.fori_loop(0, _EW // _C, chunk_body, 0)
    plsc.subcore_barrier()

    pltpu.sync_copy(acc.at[pl.ds(s * _RPT, _RPT)],
                    out_hbm.at[c, pl.ds(s * _RPT, _RPT)])

    @pl.when(s == _NTILE - 1)
    def _write_tail():
        pltpu.sync_copy(acc.at[pl.ds(_NTILE * _RPT, _TAIL)],
                        out_hbm.at[c, pl.ds(_NTILE * _RPT, _TAIL)])


@functools.cache
def _make_spmm(D):
    mesh = plsc.VectorSubcoreMesh(core_axis_name="c", subcore_axis_name="s")
    return pl.kernel(
        functools.partial(_spmm_body, D),
        out_type=jax.ShapeDtypeStruct((_NSC, _N, D), jnp.float32),
        mesh=mesh,
        scratch_types=[
            pltpu.VMEM((_C,), jnp.int32),
            pltpu.VMEM((_C,), jnp.int32),
            pltpu.VMEM((_C,), jnp.float32),
            pltpu.VMEM((_C, D), jnp.float32),
            pltpu.VMEM((_ZC, D), jnp.float32),
            pltpu.VMEM_SHARED((_N, D), jnp.float32),
            pltpu.SemaphoreType.DMA,
        ],
        name=f"gcn_spmm_d{D}",
    )


def _matmul_body(x_ref, w_ref, o_ref):
    o_ref[...] = jnp.dot(x_ref[...], w_ref[...],
                         preferred_element_type=jnp.float32)


def _fused_body(p0_ref, p1_ref, b_ref, w_ref, o_ref):
    h = jnp.maximum(p0_ref[...] + p1_ref[...] + b_ref[...], 0.0)
    o_ref[...] = jnp.dot(h, w_ref[...], preferred_element_type=jnp.float32)


def _combine_relu_body(p0_ref, p1_ref, b_ref, o_ref):
    o_ref[...] = jnp.maximum(p0_ref[...] + p1_ref[...] + b_ref[...], 0.0)


def _final_body(p0_ref, p1_ref, w_ref, b_ref, o_ref):
    o_ref[...] = jnp.dot(p0_ref[...] + p1_ref[...], w_ref[...],
                         preferred_element_type=jnp.float32) + b_ref[...]


_BLK = 1000


def _matmul(x, W):
    K, M = W.shape
    return pl.pallas_call(
        _matmul_body,
        grid=(_N // _BLK,),
        in_specs=[
            pl.BlockSpec((_BLK, K), lambda i: (i, 0)),
            pl.BlockSpec((K, M), lambda i: (0, 0)),
        ],
        out_specs=pl.BlockSpec((_BLK, M), lambda i: (i, 0)),
        out_shape=jax.ShapeDtypeStruct((_N, M), jnp.float32),
    )(x, W)


def _fused(p0, p1, b, W):
    K, M = W.shape
    return pl.pallas_call(
        _fused_body,
        grid=(_N // _BLK,),
        in_specs=[
            pl.BlockSpec((_BLK, K), lambda i: (i, 0)),
            pl.BlockSpec((_BLK, K), lambda i: (i, 0)),
            pl.BlockSpec((1, K), lambda i: (0, 0)),
            pl.BlockSpec((K, M), lambda i: (0, 0)),
        ],
        out_specs=pl.BlockSpec((_BLK, M), lambda i: (i, 0)),
        out_shape=jax.ShapeDtypeStruct((_N, M), jnp.float32),
    )(p0, p1, b.reshape(1, K), W)


def _combine_relu(p0, p1, b):
    M = p0.shape[1]
    return pl.pallas_call(
        _combine_relu_body,
        grid=(_N // _BLK,),
        in_specs=[
            pl.BlockSpec((_BLK, M), lambda i: (i, 0)),
            pl.BlockSpec((_BLK, M), lambda i: (i, 0)),
            pl.BlockSpec((1, M), lambda i: (0, 0)),
        ],
        out_specs=pl.BlockSpec((_BLK, M), lambda i: (i, 0)),
        out_shape=jax.ShapeDtypeStruct((_N, M), jnp.float32),
    )(p0, p1, b.reshape(1, M))


def _final(p0, p1, W, b):
    K, M = W.shape
    return pl.pallas_call(
        _final_body,
        grid=(_N // _BLK,),
        in_specs=[
            pl.BlockSpec((_BLK, K), lambda i: (i, 0)),
            pl.BlockSpec((_BLK, K), lambda i: (i, 0)),
            pl.BlockSpec((K, M), lambda i: (0, 0)),
            pl.BlockSpec((1, M), lambda i: (0, 0)),
        ],
        out_specs=pl.BlockSpec((_BLK, M), lambda i: (i, 0)),
        out_shape=jax.ShapeDtypeStruct((_N, M), jnp.float32),
    )(p0, p1, W, b.reshape(1, M))


def kernel(x, edge_index, edge_weight, W0, b0, W1, b1, W2, b2):
    row = edge_index[0].astype(jnp.int32)
    col = edge_index[1].astype(jnp.int32)
    w = edge_weight.astype(jnp.float32)

    spmm128 = _make_spmm(128)

    t0 = _matmul(x, W0)
    p0 = spmm128(t0, col, row, w)
    t1 = _fused(p0[0], p0[1], b0, W1)
    p1 = spmm128(t1, col, row, w)
    t2 = _combine_relu(p1[0], p1[1], b1)
    p2 = spmm128(t2, col, row, w)
    return _final(p2[0], p2[1], W2, b2)
